# baseline (speedup 1.0000x reference)
import functools
import jax
import jax.numpy as jnp
from jax import lax
from jax.experimental import pallas as pl
from jax.experimental.pallas import tpu as pltpu
from jax.experimental.pallas import tpu_sc as plsc

_VARIANCE = 1e-11
_NC = 2
_NSUB = 16
_L = 16
_R = 8
_SC_ROWS = 256

_GATHER_DNUMS = lax.GatherDimensionNumbers(
    offset_dims=(), collapsed_slice_dims=(0,), start_index_map=(0,))


def _xlane_sum(v):
    base = lax.broadcasted_iota(jnp.int32, (_L,), 0)
    for sh in (1, 2, 4, 8):
        idx = lax.bitwise_xor(base, jnp.int32(sh))
        g = lax.gather(v, idx[:, None], _GATHER_DNUMS, (1,),
                       mode=lax.GatherScatterMode.PROMISE_IN_BOUNDS)
        v = v + g
    return v


def _rsqrt16(var):
    i = lax.bitcast_convert_type(var, jnp.int32)
    i = jnp.int32(0x5F3759DF) - lax.shift_right_logical(i, 1)
    y = lax.bitcast_convert_type(i, jnp.float32)
    for _ in range(3):
        y = y * (1.5 - 0.5 * var * y * y)
    return y


def _sc_body(x_hbm, pos_hbm, g_hbm, bt_hbm, out_hbm,
             xbuf, pebuf, gbuf, btbuf, xsem, pesem, osem):
    S, B, D = x_hbm.shape
    nvec = D // _L
    wid = lax.axis_index("s") * _NC + lax.axis_index("c")
    rows_per_worker = S // (_NC * _NSUB)
    nchunks = rows_per_worker // _R
    row0 = wid * rows_per_worker

    pltpu.sync_copy(g_hbm, gbuf)
    pltpu.sync_copy(bt_hbm, btbuf)

    def start_in(k, slot):
        r = row0 + k * _R
        pltpu.make_async_copy(x_hbm.at[pl.ds(r, _R)], xbuf.at[slot],
                              xsem.at[slot]).start()
        pltpu.make_async_copy(pos_hbm.at[pl.ds(r, _R)], pebuf.at[slot],
                              pesem.at[slot]).start()

    def wait_in(slot):
        pltpu.make_async_copy(x_hbm.at[pl.ds(0, _R)], xbuf.at[slot],
                              xsem.at[slot]).wait()
        pltpu.make_async_copy(pos_hbm.at[pl.ds(0, _R)], pebuf.at[slot],
                              pesem.at[slot]).wait()

    def start_out(k, slot):
        r = row0 + k * _R
        pltpu.make_async_copy(xbuf.at[slot], out_hbm.at[pl.ds(r, _R)],
                              osem.at[slot]).start()

    def wait_out(slot):
        pltpu.make_async_copy(xbuf.at[slot], out_hbm.at[pl.ds(0, _R)],
                              osem.at[slot]).wait()

    def row_body(si, slot):
        accs = [(jnp.zeros((_L,), jnp.float32),
                 jnp.zeros((_L,), jnp.float32)) for _ in range(B)]
        for j in range(nvec):
            o = j * _L
            pe_v = pebuf[slot, si, pl.ds(o, _L)]
            for bi in range(B):
                w = xbuf[slot, si, bi, pl.ds(o, _L)] + pe_v
                xbuf[slot, si, bi, pl.ds(o, _L)] = w
                s_acc, q_acc = accs[bi]
                accs[bi] = (s_acc + w, q_acc + w * w)
        scale = []
        for bi in range(B):
            s_acc, q_acc = accs[bi]
            mean = _xlane_sum(s_acc) * (1.0 / D)
            msq = _xlane_sum(q_acc) * (1.0 / D)
            inv = _rsqrt16(msq - mean * mean + _VARIANCE)
            scale.append((inv, -mean * inv))
        for j in range(nvec):
            o = j * _L
            g_v = gbuf[pl.ds(o, _L)]
            bt_v = btbuf[pl.ds(o, _L)]
            for bi in range(B):
                inv, c = scale[bi]
                w = xbuf[slot, si, bi, pl.ds(o, _L)]
                xbuf[slot, si, bi, pl.ds(o, _L)] = (w * inv + c) * g_v + bt_v

    def chunk_body(k, _):
        slot = lax.rem(k, 2)
        nslot = 1 - slot

        @pl.when(jnp.logical_and(k + 1 < nchunks, k >= 1))
        def _():
            wait_out(nslot)

        @pl.when(k + 1 < nchunks)
        def _():
            start_in(k + 1, nslot)

        wait_in(slot)
        lax.fori_loop(0, _R, lambda si, c: (row_body(si, slot), c)[1], 0)
        start_out(k, slot)
        return 0

    start_in(0, 0)
    lax.fori_loop(0, nchunks, chunk_body, 0)
    if nchunks >= 2:
        wait_out((nchunks - 2) % 2)
    wait_out((nchunks - 1) % 2)


def _sc_kernel(x, pos_table, gamma, beta):
    S, B, D = x.shape
    mesh = plsc.VectorSubcoreMesh(core_axis_name="c", subcore_axis_name="s")
    sc_call = functools.partial(
        pl.kernel,
        mesh=mesh,
        out_type=jax.ShapeDtypeStruct((S, B, D), jnp.float32),
        scratch_types=[
            pltpu.VMEM((2, _R, B, D), jnp.float32),
            pltpu.VMEM((2, _R, D), jnp.float32),
            pltpu.VMEM((D,), jnp.float32),
            pltpu.VMEM((D,), jnp.float32),
            pltpu.SemaphoreType.DMA((2,)),
            pltpu.SemaphoreType.DMA((2,)),
            pltpu.SemaphoreType.DMA((2,)),
        ],
    )(_sc_body)
    return sc_call(x, pos_table, gamma, beta)


def _ln_body(x_ref, pos_ref, gamma_ref, beta_ref, out_ref):
    BS, B, D = x_ref.shape
    pe = pos_ref[...]
    g = gamma_ref[0][None, :]
    bt = beta_ref[0][None, :]
    for b in range(B):
        v = x_ref[:, b, :] + pe
        u = jnp.mean(v, axis=-1, keepdims=True)
        q = jnp.mean(v * v, axis=-1, keepdims=True)
        inv = jax.lax.rsqrt(q - u * u + _VARIANCE)
        out_ref[:, b, :] = (v * inv - u * inv) * g + bt


def _tc_kernel(x, pos_table, gamma, beta):
    S, B, D = x.shape
    BS = 256
    grid = (S // BS,)
    gamma2 = gamma.reshape(1, D)
    beta2 = beta.reshape(1, D)
    return pl.pallas_call(
        _ln_body,
        grid=grid,
        in_specs=[
            pl.BlockSpec((BS, B, D), lambda i: (i, 0, 0)),
            pl.BlockSpec((BS, D), lambda i: (i, 0)),
            pl.BlockSpec((1, D), lambda i: (0, 0)),
            pl.BlockSpec((1, D), lambda i: (0, 0)),
        ],
        out_specs=pl.BlockSpec((BS, B, D), lambda i: (i, 0, 0)),
        out_shape=jax.ShapeDtypeStruct((S, B, D), x.dtype),
    )(x, pos_table, gamma2, beta2)


def kernel(x, pos_table, gamma, beta):
    S, B, D = x.shape
    s_tc = S - _SC_ROWS
    out_tc = _tc_kernel(x[:s_tc], pos_table[:s_tc], gamma, beta)
    out_sc = _sc_kernel(x[s_tc:], pos_table[s_tc:], gamma, beta)
    return jnp.concatenate([out_tc, out_sc], axis=0)


# speedup vs baseline: 3.6461x; 3.6461x over previous
import jax
import jax.numpy as jnp
from jax.experimental import pallas as pl

_VARIANCE = 1e-11


def _ln_body(x_ref, pos_ref, gamma_ref, beta_ref, out_ref):
    BS, B, D = x_ref.shape
    pe = pos_ref[...]
    g = gamma_ref[0][None, :]
    bt = beta_ref[0][None, :]
    for b in range(B):
        v = x_ref[:, b, :] + pe
        u = jnp.mean(v, axis=-1, keepdims=True)
        q = jnp.mean(v * v, axis=-1, keepdims=True)
        inv = jax.lax.rsqrt(q - u * u + _VARIANCE)
        out_ref[:, b, :] = (v * inv - u * inv) * g + bt


def kernel(x, pos_table, gamma, beta):
    S, B, D = x.shape
    BS = 256
    grid = (S // BS,)
    gamma2 = gamma.reshape(1, D)
    beta2 = beta.reshape(1, D)
    return pl.pallas_call(
        _ln_body,
        grid=grid,
        in_specs=[
            pl.BlockSpec((BS, B, D), lambda i: (i, 0, 0)),
            pl.BlockSpec((BS, D), lambda i: (i, 0)),
            pl.BlockSpec((1, D), lambda i: (0, 0)),
            pl.BlockSpec((1, D), lambda i: (0, 0)),
        ],
        out_specs=pl.BlockSpec((BS, B, D), lambda i: (i, 0, 0)),
        out_shape=jax.ShapeDtypeStruct((S, B, D), x.dtype),
    )(x, pos_table, gamma2, beta2)
